# baseline (speedup 1.0000x reference)
import functools

import jax
import jax.numpy as jnp
from jax import lax
from jax.experimental import pallas as pl
from jax.experimental.pallas import tpu as pltpu
from jax.experimental.pallas import tpu_sc as plsc

QN = 2048
GROUPS = (1280, 768)
D = 768
CN = 100000
CB = 2048
CPAD = 100352
NSTEP = CPAD // CB
NBLK = CPAD // 128
NBLK_REAL = 782
TOPK = 100
NEG = -1e30

NC = 2
NS = 16
NW = NC * NS
BCAP = 128
CCAP = 256


def _mm_body(qg, q_ref, c_ref, sim_ref, m_ref):
    s = lax.dot_general(
        q_ref[...], c_ref[...],
        dimension_numbers=(((1,), (1,)), ((), ())),
        preferred_element_type=jnp.float32,
    )
    step = pl.program_id(0)

    def store(sv):
        s3 = sv.reshape(qg, CB // 128, 128)
        sim_ref[...] = s3
        m_ref[...] = jnp.max(s3, axis=2).T

    @pl.when(step < NSTEP - 1)
    def _():
        store(s)

    @pl.when(step == NSTEP - 1)
    def _():
        col = step * CB + lax.broadcasted_iota(jnp.int32, (qg, CB), 1)
        store(jnp.where(col < CN, s, NEG))


def _matmul(q, c, qg):
    return pl.pallas_call(
        functools.partial(_mm_body, qg),
        grid=(NSTEP,),
        in_specs=[
            pl.BlockSpec((qg, D), lambda i: (0, 0)),
            pl.BlockSpec((CB, D), lambda i: (i, 0)),
        ],
        out_specs=[
            pl.BlockSpec((qg, CB // 128, 128), lambda i: (0, i, 0)),
            pl.BlockSpec((CB // 128, qg), lambda i: (i, 0)),
        ],
        out_shape=[
            jax.ShapeDtypeStruct((qg, NBLK, 128), jnp.float32),
            jax.ShapeDtypeStruct((NBLK, qg), jnp.float32),
        ],
    )(q, c)


def _thr_body(qg, m_ref, t_ref, m3_ref):
    m = m_ref[...]
    blk = lax.broadcasted_iota(jnp.int32, (NBLK, qg), 0)
    real = blk < NBLK_REAL
    lo = jnp.min(jnp.where(real, m, jnp.inf), axis=0, keepdims=True)
    hi = jnp.max(m, axis=0, keepdims=True) + 1.0

    def body(_, carry):
        lo, hi = carry
        mid = 0.5 * (lo + hi)
        cnt = jnp.sum((m >= mid).astype(jnp.int32), axis=0, keepdims=True)
        pred = cnt >= TOPK
        return jnp.where(pred, mid, lo), jnp.where(pred, hi, mid)

    lo, hi = lax.fori_loop(0, 28, body, (lo, hi))
    t_ref[...] = lo.reshape(qg // 128, 128)
    mpad = jnp.concatenate(
        [m.T, jnp.full((qg, 896 - NBLK), NEG, jnp.float32)], axis=1)
    m3_ref[...] = mpad.reshape(qg, 7, 128)


def _threshold(m, qg):
    return pl.pallas_call(
        functools.partial(_thr_body, qg),
        out_shape=[
            jax.ShapeDtypeStruct((qg // 128, 128), jnp.float32),
            jax.ShapeDtypeStruct((qg, 7, 128), jnp.float32),
        ],
    )(m)


def _sc_body(qpw, sim_hbm, thr_hbm, m_hbm, oval_hbm, oidx_hbm,
             m_v, t_v, ridA, cbA, rowsA, valb, idxb, semA):
    wid = lax.axis_index("s") * NC + lax.axis_index("c")
    q0 = wid * qpw
    pltpu.sync_copy(m_hbm.at[pl.ds(q0 * 896, qpw * 896)], m_v)
    pltpu.sync_copy(thr_hbm.at[pl.ds(q0, qpw)], t_v)
    lane = lax.iota(jnp.int32, 16)
    negs = jnp.full((16,), NEG, jnp.float32)

    def initv(i, _):
        for u in range(8):
            valb[pl.ds(i * 128 + u * 16, 16)] = negs
        return 0

    lax.fori_loop(0, qpw * 2, initv, 0)

    def popcount(msk):
        return plsc.all_reduce_population_count(msk)[0]

    def tsplat(j):
        return plsc.load_gather(t_v, [jnp.full((16,), j, jnp.int32)])

    def scan_query(j, rid_v, cb_v):
        q = q0 + j
        t_spl = tsplat(j)
        pad_rid = jnp.full((16,), q * NBLK + (NBLK - 1), jnp.int32)
        for u in range(BCAP // 16):
            rid_v[pl.ds(u * 16, 16)] = pad_rid

        def scan_g(b7, nb):
            for u in range(8):
                mv = m_v[pl.ds(j * 896 + b7 * 128 + u * 16, 16)]
                msk = mv >= t_spl
                blk = (b7 * 8 + u) * 16 + lane
                off = jnp.minimum(nb, BCAP - 16)
                plsc.store_compressed(rid_v.at[pl.ds(off, 16)],
                                      q * NBLK + blk, mask=msk)
                plsc.store_compressed(cb_v.at[pl.ds(off, 16)],
                                      blk * 128, mask=msk)
                nb = nb + popcount(msk)
            return nb

        return lax.fori_loop(0, 7, scan_g, jnp.int32(0))

    def filter_query(j, rows_v, cb_v, nb):
        t_spl = tsplat(j)
        base = j * 256

        def filt(r, cnt):
            cb_spl = plsc.load_gather(cb_v, [jnp.full((16,), r, jnp.int32)])
            for u in range(8):
                v = rows_v[r, pl.ds(u * 16, 16)]
                msk = v >= t_spl
                off = base + jnp.minimum(cnt, CCAP - 16)
                plsc.store_compressed(valb.at[pl.ds(off, 16)], v, mask=msk)
                col = cb_spl + u * 16 + lane
                plsc.store_compressed(idxb.at[pl.ds(off, 16)], col, mask=msk)
                cnt = cnt + popcount(msk)
            return cnt

        lax.fori_loop(0, nb, filt, jnp.int32(0))

    def body(i, _):
        nbA = scan_query(i, ridA, cbA)
        pltpu.async_copy(sim_hbm.at[ridA], rowsA, semA).wait()
        filter_query(i, rowsA, cbA, nbA)
        return 0

    lax.fori_loop(0, qpw, body, 0)

    pltpu.sync_copy(valb, oval_hbm.at[pl.ds(q0 * 256, qpw * 256)])
    pltpu.sync_copy(idxb, oidx_hbm.at[pl.ds(q0 * 256, qpw * 256)])


def _sc_select(sim, thr, m3, qg):
    qpw = qg // NW
    fn = functools.partial(
        pl.kernel,
        mesh=plsc.VectorSubcoreMesh(core_axis_name="c", subcore_axis_name="s"),
        compiler_params=pltpu.CompilerParams(needs_layout_passes=False),
        out_type=[
            jax.ShapeDtypeStruct((qg * 256,), jnp.float32),
            jax.ShapeDtypeStruct((qg * 256,), jnp.int32),
        ],
        scratch_types=[
            pltpu.VMEM((qpw * 896,), jnp.float32),
            pltpu.VMEM((qpw,), jnp.float32),
            pltpu.VMEM((BCAP,), jnp.int32),
            pltpu.VMEM((BCAP,), jnp.int32),
            pltpu.VMEM((BCAP, 128), jnp.float32),
            pltpu.VMEM((qpw * 256,), jnp.float32),
            pltpu.VMEM((qpw * 256,), jnp.int32),
            pltpu.SemaphoreType.DMA,
        ],
    )(functools.partial(_sc_body, qpw))
    return fn(sim, thr, m3)


def _topk_body(qg, v_ref, i_ref, ov_ref, oi_ref, s_ref):
    s_ref[...] = v_ref[...].reshape(qg, CCAP)
    idx = i_ref[...].reshape(qg, CCAP)
    kcol = lax.broadcasted_iota(jnp.int32, (qg, 128), 1)
    ov_ref[...] = jnp.zeros((qg, 128), jnp.float32)
    oi_ref[...] = jnp.zeros((qg, 128), jnp.int32)

    def body(k, _):
        vv = s_ref[...]
        m = jnp.max(vv, axis=1, keepdims=True)
        ism = vv >= m
        isel = jnp.min(jnp.where(ism, idx, jnp.int32(2147483647)),
                       axis=1, keepdims=True)
        ov_ref[...] = jnp.where(kcol == k, m, ov_ref[...])
        oi_ref[...] = jnp.where(kcol == k, isel, oi_ref[...])
        s_ref[...] = jnp.where(ism & (idx == isel), NEG, vv)
        return 0

    lax.fori_loop(0, TOPK, body, 0)


def _final_topk(cval, cidx, qg):
    return pl.pallas_call(
        functools.partial(_topk_body, qg),
        scratch_shapes=[pltpu.VMEM((qg, CCAP), jnp.float32)],
        out_shape=[
            jax.ShapeDtypeStruct((qg, 128), jnp.float32),
            jax.ShapeDtypeStruct((qg, 128), jnp.int32),
        ],
    )(cval, cidx)


def kernel(query_vectors, corpus_vectors):
    ovs, ois = [], []
    off = 0
    for qg in GROUPS:
        q = lax.slice_in_dim(query_vectors, off, off + qg, axis=0)
        off += qg
        sim, m = _matmul(q, corpus_vectors, qg)
        thr, m3 = _threshold(m, qg)
        cval, cidx = _sc_select(sim.reshape(qg * NBLK, 128),
                                thr.reshape(qg), m3.reshape(qg * 896), qg)
        ov, oi = _final_topk(cval.reshape(qg, 2, 128),
                             cidx.reshape(qg, 2, 128), qg)
        ovs.append(ov[:, :TOPK])
        ois.append(oi[:, :TOPK])
    return (jnp.concatenate(ovs, axis=0), jnp.concatenate(ois, axis=0))


# speedup vs baseline: 1.0460x; 1.0460x over previous
import functools

import jax
import jax.numpy as jnp
from jax import lax
from jax.experimental import pallas as pl
from jax.experimental.pallas import tpu as pltpu
from jax.experimental.pallas import tpu_sc as plsc

QN = 2048
GROUPS = (1024, 1024)
D = 768
CN = 100000
CB = 2048
CPAD = 100352
NSTEP = CPAD // CB
NBLK = CPAD // 128
NBLK_REAL = 782
TOPK = 100
NEG = -1e30

NC = 2
NS = 16
NW = NC * NS
BCAP = 128
CCAP = 256


def _mm_body(qg, q_ref, c_ref, sim_ref, m_ref):
    s = lax.dot_general(
        q_ref[...], c_ref[...],
        dimension_numbers=(((1,), (1,)), ((), ())),
        preferred_element_type=jnp.float32,
    )
    step = pl.program_id(0)

    def store(sv):
        s3 = sv.reshape(qg, CB // 128, 128)
        sim_ref[...] = s3
        m_ref[...] = jnp.max(s3, axis=2).T

    @pl.when(step < NSTEP - 1)
    def _():
        store(s)

    @pl.when(step == NSTEP - 1)
    def _():
        col = step * CB + lax.broadcasted_iota(jnp.int32, (qg, CB), 1)
        store(jnp.where(col < CN, s, NEG))


def _matmul(q, c, qg):
    return pl.pallas_call(
        functools.partial(_mm_body, qg),
        grid=(NSTEP,),
        in_specs=[
            pl.BlockSpec((qg, D), lambda i: (0, 0)),
            pl.BlockSpec((CB, D), lambda i: (i, 0)),
        ],
        out_specs=[
            pl.BlockSpec((qg, CB // 128, 128), lambda i: (0, i, 0)),
            pl.BlockSpec((CB // 128, qg), lambda i: (i, 0)),
        ],
        out_shape=[
            jax.ShapeDtypeStruct((qg, NBLK, 128), jnp.float32),
            jax.ShapeDtypeStruct((NBLK, qg), jnp.float32),
        ],
    )(q, c)


def _thr_body(qg, m_ref, t_ref, m3_ref):
    m = m_ref[...]
    blk = lax.broadcasted_iota(jnp.int32, (NBLK, qg), 0)
    real = blk < NBLK_REAL
    lo = jnp.min(jnp.where(real, m, jnp.inf), axis=0, keepdims=True)
    hi = jnp.max(m, axis=0, keepdims=True) + 1.0

    def body(_, carry):
        lo, hi = carry
        mid = 0.5 * (lo + hi)
        cnt = jnp.sum((m >= mid).astype(jnp.int32), axis=0, keepdims=True)
        pred = cnt >= TOPK
        return jnp.where(pred, mid, lo), jnp.where(pred, hi, mid)

    lo, hi = lax.fori_loop(0, 28, body, (lo, hi))
    t_ref[...] = lo.reshape(qg // 128, 128)
    mpad = jnp.concatenate(
        [m.T, jnp.full((qg, 896 - NBLK), NEG, jnp.float32)], axis=1)
    m3_ref[...] = mpad.reshape(qg, 7, 128)


def _threshold(m, qg):
    return pl.pallas_call(
        functools.partial(_thr_body, qg),
        out_shape=[
            jax.ShapeDtypeStruct((qg // 128, 128), jnp.float32),
            jax.ShapeDtypeStruct((qg, 7, 128), jnp.float32),
        ],
    )(m)


def _sc_body(qpw, sim_hbm, thr_hbm, m_hbm, oval_hbm, oidx_hbm,
             m_v, t_v, ridA, cbA, rowsA, valb, idxb, semA):
    wid = lax.axis_index("s") * NC + lax.axis_index("c")
    q0 = wid * qpw
    pltpu.sync_copy(m_hbm.at[pl.ds(q0 * 896, qpw * 896)], m_v)
    pltpu.sync_copy(thr_hbm.at[pl.ds(q0, qpw)], t_v)
    lane = lax.iota(jnp.int32, 16)
    negs = jnp.full((16,), NEG, jnp.float32)

    def initv(i, _):
        for u in range(8):
            valb[pl.ds(i * 128 + u * 16, 16)] = negs
        return 0

    lax.fori_loop(0, qpw * 2, initv, 0)

    def popcount(msk):
        return plsc.all_reduce_population_count(msk)[0]

    def tsplat(j):
        return plsc.load_gather(t_v, [jnp.full((16,), j, jnp.int32)])

    def scan_query(j, rid_v, cb_v):
        q = q0 + j
        t_spl = tsplat(j)
        pad_rid = jnp.full((16,), q * NBLK + (NBLK - 1), jnp.int32)
        for u in range(BCAP // 16):
            rid_v[pl.ds(u * 16, 16)] = pad_rid

        def scan_g(b7, nb):
            for u in range(8):
                mv = m_v[pl.ds(j * 896 + b7 * 128 + u * 16, 16)]
                msk = mv >= t_spl
                blk = (b7 * 8 + u) * 16 + lane
                off = jnp.minimum(nb, BCAP - 16)
                plsc.store_compressed(rid_v.at[pl.ds(off, 16)],
                                      q * NBLK + blk, mask=msk)
                plsc.store_compressed(cb_v.at[pl.ds(off, 16)],
                                      blk * 128, mask=msk)
                nb = nb + popcount(msk)
            return nb

        return lax.fori_loop(0, 7, scan_g, jnp.int32(0))

    def filter_query(j, rows_v, cb_v, nb):
        t_spl = tsplat(j)
        base = j * 256

        def filt(r, cnt):
            cb_spl = plsc.load_gather(cb_v, [jnp.full((16,), r, jnp.int32)])
            for u in range(8):
                v = rows_v[r, pl.ds(u * 16, 16)]
                msk = v >= t_spl
                off = base + jnp.minimum(cnt, CCAP - 16)
                plsc.store_compressed(valb.at[pl.ds(off, 16)], v, mask=msk)
                col = cb_spl + u * 16 + lane
                plsc.store_compressed(idxb.at[pl.ds(off, 16)], col, mask=msk)
                cnt = cnt + popcount(msk)
            return cnt

        lax.fori_loop(0, nb, filt, jnp.int32(0))

    def body(i, _):
        nbA = scan_query(i, ridA, cbA)
        pltpu.async_copy(sim_hbm.at[ridA], rowsA, semA).wait()
        filter_query(i, rowsA, cbA, nbA)
        return 0

    lax.fori_loop(0, qpw, body, 0)

    pltpu.sync_copy(valb, oval_hbm.at[pl.ds(q0 * 256, qpw * 256)])
    pltpu.sync_copy(idxb, oidx_hbm.at[pl.ds(q0 * 256, qpw * 256)])


def _sc_select(sim, thr, m3, qg):
    qpw = qg // NW
    fn = functools.partial(
        pl.kernel,
        mesh=plsc.VectorSubcoreMesh(core_axis_name="c", subcore_axis_name="s"),
        compiler_params=pltpu.CompilerParams(needs_layout_passes=False),
        out_type=[
            jax.ShapeDtypeStruct((qg * 256,), jnp.float32),
            jax.ShapeDtypeStruct((qg * 256,), jnp.int32),
        ],
        scratch_types=[
            pltpu.VMEM((qpw * 896,), jnp.float32),
            pltpu.VMEM((qpw,), jnp.float32),
            pltpu.VMEM((BCAP,), jnp.int32),
            pltpu.VMEM((BCAP,), jnp.int32),
            pltpu.VMEM((BCAP, 128), jnp.float32),
            pltpu.VMEM((qpw * 256,), jnp.float32),
            pltpu.VMEM((qpw * 256,), jnp.int32),
            pltpu.SemaphoreType.DMA,
        ],
    )(functools.partial(_sc_body, qpw))
    return fn(sim, thr, m3)


def _topk_body(qg, v_ref, i_ref, ov_ref, oi_ref, s_ref):
    s_ref[...] = v_ref[...].reshape(qg, CCAP)
    idx = i_ref[...].reshape(qg, CCAP)
    kcol = lax.broadcasted_iota(jnp.int32, (qg, 128), 1)
    ov_ref[...] = jnp.zeros((qg, 128), jnp.float32)
    oi_ref[...] = jnp.zeros((qg, 128), jnp.int32)

    def body(k, _):
        vv = s_ref[...]
        m = jnp.max(vv, axis=1, keepdims=True)
        ism = vv >= m
        isel = jnp.min(jnp.where(ism, idx, jnp.int32(2147483647)),
                       axis=1, keepdims=True)
        ov_ref[...] = jnp.where(kcol == k, m, ov_ref[...])
        oi_ref[...] = jnp.where(kcol == k, isel, oi_ref[...])
        s_ref[...] = jnp.where(ism & (idx == isel), NEG, vv)
        return 0

    lax.fori_loop(0, TOPK, body, 0)


def _final_topk(cval, cidx, qg):
    return pl.pallas_call(
        functools.partial(_topk_body, qg),
        scratch_shapes=[pltpu.VMEM((qg, CCAP), jnp.float32)],
        out_shape=[
            jax.ShapeDtypeStruct((qg, 128), jnp.float32),
            jax.ShapeDtypeStruct((qg, 128), jnp.int32),
        ],
    )(cval, cidx)


def kernel(query_vectors, corpus_vectors):
    ovs, ois = [], []
    off = 0
    for qg in GROUPS:
        q = lax.slice_in_dim(query_vectors, off, off + qg, axis=0)
        off += qg
        sim, m = _matmul(q, corpus_vectors, qg)
        thr, m3 = _threshold(m, qg)
        cval, cidx = _sc_select(sim.reshape(qg * NBLK, 128),
                                thr.reshape(qg), m3.reshape(qg * 896), qg)
        ov, oi = _final_topk(cval.reshape(qg, 2, 128),
                             cidx.reshape(qg, 2, 128), qg)
        ovs.append(ov[:, :TOPK])
        ois.append(oi[:, :TOPK])
    return (jnp.concatenate(ovs, axis=0), jnp.concatenate(ois, axis=0))
